# baseline (speedup 1.0000x reference)
import jax
import jax.numpy as jnp
from jax.experimental import pallas as pl
from jax.experimental.pallas import tpu as pltpu

NSEG = 16


def _body(idx_ref, w_ref, xa_ref, xb_ref, out_ref):
    H = xa_ref.shape[2]
    halves = (xa_ref[0, 0], xb_ref[0, 0])

    idx = idx_ref[...]
    rows = jax.lax.broadcasted_iota(jnp.int32, (NSEG, 2 * H), 0)
    oh_t = (rows == idx[None, :]).astype(jnp.float32)
    oh_h = (oh_t[:, :H], oh_t[:, H:])

    cnt = jnp.sum(oh_t, axis=1)
    inv = 1.0 / jnp.clip(cnt, 1.0, None)

    sums = sum(
        jax.lax.dot_general(o, xh, (((1,), (0,)), ((), ())),
                            preferred_element_type=jnp.float32)
        for o, xh in zip(oh_h, halves))
    gc = jnp.tanh(
        jax.lax.dot_general(sums * inv[:, None], w_ref[...],
                            (((1,), (0,)), ((), ())),
                            preferred_element_type=jnp.float32))
    gcb = gc.astype(jnp.bfloat16)

    acc = jnp.zeros((NSEG, xa_ref.shape[3]), jnp.float32)
    for o, xh in zip(oh_h, halves):
        xb = xh.astype(jnp.bfloat16)
        scores = jax.lax.dot_general(
            gcb, xb, (((1,), (1,)), ((), ())),
            preferred_element_type=jnp.float32)
        gate_c = (jax.nn.sigmoid(
            jnp.sum(scores * o, axis=0, keepdims=True)) - 0.5)
        ohg = (o * gate_c).astype(jnp.bfloat16)
        acc = acc + jax.lax.dot_general(
            ohg, xb, (((1,), (0,)), ((), ())),
            preferred_element_type=jnp.float32)

    out_ref[0] = (acc + 0.5 * sums) * inv[:, None]


@jax.jit
def kernel(x, batch_index, weight):
    J, F, C = x.shape
    idx = batch_index.astype(jnp.int32)
    H = F // 2
    xh = x.reshape(J, 2, H, C)

    out = pl.pallas_call(
        _body,
        grid=(J,),
        in_specs=[
            pl.BlockSpec((F,), lambda j: (0,)),
            pl.BlockSpec((C, C), lambda j: (0, 0)),
            pl.BlockSpec((1, 1, H, C), lambda j: (j, 0, 0, 0)),
            pl.BlockSpec((1, 1, H, C), lambda j: (j, 1, 0, 0)),
        ],
        out_specs=pl.BlockSpec((1, NSEG, C), lambda j: (j, 0, 0)),
        out_shape=jax.ShapeDtypeStruct((J, NSEG, C), jnp.float32),
    )(idx, weight, xh, xh)
    return out


# speedup vs baseline: 1.0258x; 1.0258x over previous
import jax
import jax.numpy as jnp
from jax.experimental import pallas as pl

NSEG = 16


def _body(idx_ref, w_ref, x_ref, out_ref):
    F = x_ref.shape[1]
    xj = x_ref[0]

    idx = idx_ref[...]
    rows = jax.lax.broadcasted_iota(jnp.int32, (NSEG, F), 0)
    oh_t = (rows == idx[None, :]).astype(jnp.float32)

    cnt = jnp.sum(oh_t, axis=1)
    inv = 1.0 / jnp.clip(cnt, 1.0, None)

    sums = jax.lax.dot_general(
        oh_t, xj, (((1,), (0,)), ((), ())),
        preferred_element_type=jnp.float32)
    gc = jnp.tanh(
        jax.lax.dot_general(sums * inv[:, None], w_ref[...],
                            (((1,), (0,)), ((), ())),
                            preferred_element_type=jnp.float32))

    xb = xj.astype(jnp.bfloat16)
    scores = jax.lax.dot_general(
        gc.astype(jnp.bfloat16), xb, (((1,), (1,)), ((), ())),
        preferred_element_type=jnp.float32)
    gate_c = (jax.nn.sigmoid(
        jnp.sum(scores * oh_t, axis=0, keepdims=True)) - 0.5)
    ohg = (oh_t * gate_c).astype(jnp.bfloat16)
    acc = jax.lax.dot_general(
        ohg, xb, (((1,), (0,)), ((), ())),
        preferred_element_type=jnp.float32)

    out_ref[0] = (acc + 0.5 * sums) * inv[:, None]


@jax.jit
def kernel(x, batch_index, weight):
    J, F, C = x.shape
    idx = batch_index.astype(jnp.int32)

    out = pl.pallas_call(
        _body,
        grid=(J,),
        in_specs=[
            pl.BlockSpec((F,), lambda j: (0,)),
            pl.BlockSpec((C, C), lambda j: (0, 0)),
            pl.BlockSpec((1, F, C), lambda j: (j, 0, 0)),
        ],
        out_specs=pl.BlockSpec((1, NSEG, C), lambda j: (j, 0, 0)),
        out_shape=jax.ShapeDtypeStruct((J, NSEG, C), jnp.float32),
    )(idx, weight, x)
    return out
